# baseline (speedup 1.0000x reference)
import functools

import jax
import jax.numpy as jnp
from jax import lax
from jax.experimental import pallas as pl
from jax.experimental.pallas import tpu as pltpu
from jax.experimental.pallas import tpu_sc as plsc

N = 50000
BATCH = 32
NUM_CLASSES = 256
E = 800000

NP = 50176
ROWS_PER_TILE = NP // 16
ZCH = ROWS_PER_TILE // 4

NTILES = 32
CHUNK = 128
KC = 2
G = 100
G4 = G // 4
EPT = G * KC * CHUNK
EPAD = EPT * NTILES

_f32 = jnp.float32


def _edge_body(xt, srcm, dstm, z2d, z1d, ones,
               sums_out, cnts0_out, cnts1_out,
               sums_sp, cnts_sp, src_v, dst_v, rows_v, ones_v,
               gsem0, gsem1, ssem0, ssem1, isem0, isem1, isem2, isem3):
    cid = lax.axis_index("c")
    sid = lax.axis_index("s")
    wid = cid * 16 + sid

    pltpu.sync_copy(ones, ones_v)

    base = sid * ROWS_PER_TILE

    def _zero(k, carry):
        pltpu.sync_copy(z2d, sums_sp.at[pl.ds(base + k * ZCH, ZCH)])
        return carry

    lax.fori_loop(0, 4, _zero, 0)
    pltpu.sync_copy(z1d, cnts_sp.at[pl.ds(base, ROWS_PER_TILE)])
    plsc.subcore_barrier()

    gsems = (gsem0, gsem1)
    ssems = (ssem0, ssem1)
    isems = (isem0, isem1, isem2, isem3)

    def stage(g, q):
        sl = pl.ds(g * KC, KC)
        pltpu.async_copy(srcm.at[wid, sl], src_v.at[q], isems[q])
        pltpu.async_copy(dstm.at[wid, sl], dst_v.at[q], isems[q])

    def drain_idx(g, q):
        sl = pl.ds(g * KC, KC)
        pltpu.make_async_copy(srcm.at[wid, sl], src_v.at[q], isems[q]).wait()
        pltpu.make_async_copy(dstm.at[wid, sl], dst_v.at[q], isems[q]).wait()

    def fire_gathers(r, q):
        for j in range(KC):
            pltpu.async_copy(xt.at[src_v.at[q, j]],
                             rows_v.at[r * KC + j], gsems[r])

    def drain_gathers(r, q):
        for j in range(KC):
            pltpu.make_async_copy(xt.at[src_v.at[q, j]],
                                  rows_v.at[r * KC + j], gsems[r]).wait()

    def fire_scatters(r, q):
        for j in range(KC):
            pltpu.async_copy(rows_v.at[r * KC + j],
                             sums_sp.at[dst_v.at[q, j]], ssems[r],
                             add=True)
            pltpu.async_copy(ones_v, cnts_sp.at[dst_v.at[q, j]],
                             ssems[r], add=True)

    def drain_scatters(r, q):
        for j in range(KC):
            pltpu.make_async_copy(rows_v.at[r * KC + j],
                                  sums_sp.at[dst_v.at[q, j]],
                                  ssems[r]).wait()
            pltpu.make_async_copy(ones_v, cnts_sp.at[dst_v.at[q, j]],
                                  ssems[r]).wait()

    stage(0, 0)
    stage(1, 1)
    drain_idx(0, 0)
    fire_gathers(0, 0)

    def body(i, carry):
        gb = 4 * i
        for u in range(4):
            g = gb + u
            r, q = u % 2, u
            rn, qn = (u + 1) % 2, (u + 1) % 4
            drain_gathers(r, q)

            if u == 0:
                @pl.when(i > 0)
                def _():
                    drain_scatters(1, 3)
            else:
                drain_scatters((u - 1) % 2, u - 1)

            if u < 2:
                stage(g + 2, u + 2)
            else:
                @pl.when(i < G4 - 1)
                def _():
                    stage(g + 2, (u + 2) % 4)

            if u < 3:
                drain_idx(g + 1, qn)
                fire_gathers(rn, qn)
            else:
                @pl.when(i < G4 - 1)
                def _():
                    drain_idx(g + 1, qn)
                    fire_gathers(rn, qn)

            fire_scatters(r, q)
        return carry

    lax.fori_loop(0, G4, body, 0)
    drain_scatters(1, 3)
    plsc.subcore_barrier()

    pltpu.sync_copy(sums_sp.at[pl.ds(base, ROWS_PER_TILE)],
                    sums_out.at[cid, pl.ds(base, ROWS_PER_TILE)])

    @pl.when(cid == 0)
    def _():
        pltpu.sync_copy(cnts_sp.at[pl.ds(base, ROWS_PER_TILE)],
                        cnts0_out.at[pl.ds(base, ROWS_PER_TILE)])

    @pl.when(cid == 1)
    def _():
        pltpu.sync_copy(cnts_sp.at[pl.ds(base, ROWS_PER_TILE)],
                        cnts1_out.at[pl.ds(base, ROWS_PER_TILE)])


@functools.cache
def _edge_pass_kernel():
    return pl.kernel(
        _edge_body,
        out_type=(jax.ShapeDtypeStruct((2, NP, BATCH), _f32),
                  jax.ShapeDtypeStruct((NP,), _f32),
                  jax.ShapeDtypeStruct((NP,), _f32)),
        mesh=plsc.VectorSubcoreMesh(core_axis_name="c", subcore_axis_name="s"),
        compiler_params=pltpu.CompilerParams(use_tc_tiling_on_sc=False),
        scratch_types=(
            pltpu.VMEM_SHARED((NP, BATCH), _f32),
            pltpu.VMEM_SHARED((NP,), _f32),
            pltpu.VMEM((4, KC, CHUNK), jnp.int32),
            pltpu.VMEM((4, KC, CHUNK), jnp.int32),
            pltpu.VMEM((2 * KC, CHUNK, BATCH), _f32),
            pltpu.VMEM((CHUNK,), _f32),
            pltpu.SemaphoreType.DMA,
            pltpu.SemaphoreType.DMA,
            pltpu.SemaphoreType.DMA,
            pltpu.SemaphoreType.DMA,
            pltpu.SemaphoreType.DMA,
            pltpu.SemaphoreType.DMA,
            pltpu.SemaphoreType.DMA,
            pltpu.SemaphoreType.DMA,
        ),
    )


NPACK = NP * BATCH // 128
NPV = NP // 128
_NB = 7
_PBLK = NPACK // _NB
_CBLK = NPV // _NB


def _expand_B():
    j = jnp.arange(128, dtype=jnp.int32)[:, None]
    m = jnp.arange(32 * 128, dtype=jnp.int32)[None, :]
    return (j == 4 * (m // 128) + (m % 128) // 32).astype(_f32)


def _expand_vec(c, b):
    cb = c.shape[0]
    e = lax.dot_general(c, b, (((1,), (0,)), ((), ())),
                        preferred_element_type=_f32)
    return e.reshape(cb * 32, 128)


_BSPEC = pl.BlockSpec((128, 32 * 128), lambda i: (0, 0))


def _scale_body(x_ref, p_ref, b_ref, o_ref):
    o_ref[...] = x_ref[...] * _expand_vec(p_ref[...], b_ref[...])


def _tc_scale(xp, p2d, bmat):
    return pl.pallas_call(
        _scale_body,
        grid=(_NB,),
        in_specs=[pl.BlockSpec((_PBLK, 128), lambda i: (i, 0)),
                  pl.BlockSpec((_CBLK, 128), lambda i: (i, 0)),
                  _BSPEC],
        out_specs=pl.BlockSpec((_PBLK, 128), lambda i: (i, 0)),
        out_shape=jax.ShapeDtypeStruct((NPACK, 128), _f32),
    )(xp, p2d, bmat)


def _merge_body(s0_ref, s1_ref, ca_ref, cb_ref, p_ref, b_ref, o_ref):
    s = s0_ref[0] + s1_ref[0]
    c = _expand_vec(ca_ref[...] + cb_ref[...], b_ref[...])
    m = s / jnp.maximum(c, 1.0)
    o_ref[...] = jnp.maximum(m, 0.0) * _expand_vec(p_ref[...], b_ref[...])


def _tc_merge(sp, ca, cb, p2d, bmat):
    return pl.pallas_call(
        _merge_body,
        grid=(_NB,),
        in_specs=[pl.BlockSpec((1, _PBLK, 128), lambda i: (0, i, 0)),
                  pl.BlockSpec((1, _PBLK, 128), lambda i: (1, i, 0)),
                  pl.BlockSpec((_CBLK, 128), lambda i: (i, 0)),
                  pl.BlockSpec((_CBLK, 128), lambda i: (i, 0)),
                  pl.BlockSpec((_CBLK, 128), lambda i: (i, 0)),
                  _BSPEC],
        out_specs=pl.BlockSpec((_PBLK, 128), lambda i: (i, 0)),
        out_shape=jax.ShapeDtypeStruct((NPACK, 128), _f32),
    )(sp, sp, ca, cb, p2d, bmat)


def _mm_body(s0_ref, s1_ref, ca_ref, cb_ref, b_ref,
             w0_ref, w1_ref, w2_ref, w3_ref,
             o_ref, acc_ref):
    k = pl.program_id(0)

    @pl.when(k == 0)
    def _():
        acc_ref[...] = jnp.zeros_like(acc_ref)

    s = s0_ref[0] + s1_ref[0]
    c = _expand_vec(ca_ref[...] + cb_ref[...], b_ref[...])
    h = jnp.maximum(s / jnp.maximum(c, 1.0), 0.0)
    acc = acc_ref[...]
    for a, w_ref in enumerate((w0_ref, w1_ref, w2_ref, w3_ref)):
        col = (a * NPACK + k * _PBLK
               + lax.broadcasted_iota(jnp.int32, (NUM_CLASSES, _PBLK), 1))
        w = jnp.where(col < N, w_ref[...], 0.0)
        acc = acc + lax.dot_general(w, h[:, 32 * a:32 * (a + 1)],
                                    (((1,), (0,)), ((), ())),
                                    preferred_element_type=_f32)
    acc_ref[...] = acc

    @pl.when(k == _NB - 1)
    def _():
        o_ref[...] = acc_ref[...]


def _tc_matmul(sp, ca, cb, bmat, W):
    w_spec = [pl.BlockSpec((NUM_CLASSES, _PBLK),
                           (lambda a: (lambda k: (0, k + a * _NB)))(a))
              for a in range(4)]
    return pl.pallas_call(
        _mm_body,
        grid=(_NB,),
        in_specs=[pl.BlockSpec((1, _PBLK, 128), lambda k: (0, k, 0)),
                  pl.BlockSpec((1, _PBLK, 128), lambda k: (1, k, 0)),
                  pl.BlockSpec((_CBLK, 128), lambda k: (k, 0)),
                  pl.BlockSpec((_CBLK, 128), lambda k: (k, 0)),
                  _BSPEC] + w_spec,
        out_specs=pl.BlockSpec((NUM_CLASSES, BATCH), lambda k: (0, 0)),
        out_shape=jax.ShapeDtypeStruct((NUM_CLASSES, BATCH), _f32),
        scratch_shapes=[pltpu.VMEM((NUM_CLASSES, BATCH), _f32)],
    )(sp, sp, ca, cb, bmat, W, W, W, W)


def _perm(idx):
    a = ((idx >= NPACK).astype(jnp.int32)
         + (idx >= 2 * NPACK).astype(jnp.int32)
         + (idx >= 3 * NPACK).astype(jnp.int32))
    return (idx - a * NPACK) * 4 + a


def _pad_edges(ei, perm_dst):
    src = ei[0]
    dst = ei[1]
    pad = EPAD - E
    pad_ar = jnp.arange(pad, dtype=jnp.int32)
    src_p = jnp.concatenate([src, pad_ar % N])
    dst_p = jnp.concatenate([dst, N + (pad_ar % 128)])
    if perm_dst:
        dst_p = _perm(dst_p)
    shape = (NTILES, G * KC, CHUNK)
    return src_p.reshape(shape), dst_p.reshape(shape)


def _vec2d(v):
    return jnp.pad(v, (0, NP - N)).reshape(NPV, 128)


def kernel(data, p1, p2, W, b, edge_index1, edge_index2):
    src1, dst1 = _pad_edges(edge_index1, perm_dst=False)
    src2, dst2 = _pad_edges(edge_index2, perm_dst=True)

    data_tp = jnp.pad(data.T, ((0, NP - N), (0, 0))).reshape(NPACK, 128)

    z2d = jnp.zeros((ZCH, BATCH), _f32)
    z1d = jnp.zeros((ROWS_PER_TILE,), _f32)
    ones = jnp.ones((CHUNK,), _f32)
    bmat = _expand_B()

    x1t = _tc_scale(data_tp, _vec2d(p1), bmat).reshape(NP, BATCH)
    sums1, c1a, c1b = _edge_pass_kernel()(x1t, src1, dst1, z2d, z1d, ones)

    x2t = _tc_merge(sums1.reshape(2, NPACK, 128),
                    c1a.reshape(NPV, 128), c1b.reshape(NPV, 128),
                    _vec2d(p2), bmat).reshape(NP, BATCH)
    sums2, c2a, c2b = _edge_pass_kernel()(x2t, src2, dst2, z2d, z1d, ones)

    out_t = _tc_matmul(sums2.reshape(2, NPACK, 128),
                       c2a.reshape(NPV, 128), c2b.reshape(NPV, 128),
                       bmat, W)
    return out_t.T + b[None, :]
